# baseline (speedup 1.0000x reference)
import functools

import jax
import jax.numpy as jnp
from jax import lax
from jax.experimental import pallas as pl
from jax.experimental.pallas import tpu as pltpu

B, R, N, D = 4, 4, 1024, 256
NTILE = 256


def _layer_body(adj_ref, x_ref, xown_ref, wr_ref, br_ref, w0_ref, b0_ref,
                out_ref, obf_ref, agg_ref, den_ref):
    r = pl.program_id(2)

    adj_blk = adj_ref[0, 0]
    x_full = x_ref[0]

    rowsum = jnp.sum(adj_blk, axis=1, keepdims=True)
    s = jnp.dot(adj_blk.astype(jnp.bfloat16), x_full,
                preferred_element_type=jnp.float32)

    wr = wr_ref[r]
    contrib = lax.dot_general(s.astype(jnp.bfloat16), wr,
                              (((1,), (1,)), ((), ())),
                              preferred_element_type=jnp.float32)
    contrib = contrib + rowsum * br_ref[r][None, :]

    @pl.when(r == 0)
    def _init():
        agg_ref[...] = contrib
        den_ref[...] = rowsum

    @pl.when(r > 0)
    def _acc():
        agg_ref[...] += contrib
        den_ref[...] += rowsum

    @pl.when(r == R - 1)
    def _finish():
        x_own = xown_ref[0]
        h0 = lax.dot_general(x_own, w0_ref[...], (((1,), (1,)), ((), ())),
                             preferred_element_type=jnp.float32)
        h0 = h0 + b0_ref[...]
        denoms = den_ref[...] + 1.0
        out = jnp.maximum((agg_ref[...] + h0) / denoms, 0.0)
        out_ref[0] = out
        obf_ref[0] = out.astype(jnp.bfloat16)


def _layer(xbf, adj, w0l, b0l, wrl, brl):
    grid = (B, N // NTILE, R)
    return pl.pallas_call(
        _layer_body,
        grid=grid,
        in_specs=[
            pl.BlockSpec((1, 1, NTILE, N), lambda b, n, r: (b, r, n, 0)),
            pl.BlockSpec((1, N, D), lambda b, n, r: (b, 0, 0)),
            pl.BlockSpec((1, NTILE, D), lambda b, n, r: (b, n, 0)),
            pl.BlockSpec((R, D, D), lambda b, n, r: (0, 0, 0)),
            pl.BlockSpec((R, D), lambda b, n, r: (0, 0)),
            pl.BlockSpec((D, D), lambda b, n, r: (0, 0)),
            pl.BlockSpec((1, D), lambda b, n, r: (0, 0)),
        ],
        out_specs=[
            pl.BlockSpec((1, NTILE, D), lambda b, n, r: (b, n, 0)),
            pl.BlockSpec((1, NTILE, D), lambda b, n, r: (b, n, 0)),
        ],
        out_shape=[
            jax.ShapeDtypeStruct((B, N, D), jnp.float32),
            jax.ShapeDtypeStruct((B, N, D), jnp.bfloat16),
        ],
        scratch_shapes=[
            pltpu.VMEM((NTILE, D), jnp.float32),
            pltpu.VMEM((NTILE, 1), jnp.float32),
        ],
    )(adj, xbf, xbf, wrl, brl, w0l, b0l)


@jax.jit
def kernel(nodes, adj, W0, b0, Wr, br):
    bf = jnp.bfloat16
    xbf = nodes.astype(bf)
    outs = []
    for l in range(W0.shape[0]):
        out, xbf = _layer(xbf, adj, W0[l].astype(bf), b0[l][None, :],
                          Wr[l].astype(bf), br[l])
        outs.append(out)
    return tuple(outs)


# speedup vs baseline: 1.1077x; 1.1077x over previous
import jax
import jax.numpy as jnp
from jax import lax
from jax.experimental import pallas as pl
from jax.experimental.pallas import tpu as pltpu

B, R, N, D = 4, 4, 1024, 256
NTILE = 256
NT = N // NTILE


def _compute(r, adj_bf, rowsum, x_full, wr_ref, br_ref, agg_ref, den_ref):
    s = jnp.dot(adj_bf, x_full, preferred_element_type=jnp.float32)
    contrib = lax.dot_general(s.astype(jnp.bfloat16), wr_ref[0, r],
                              (((1,), (1,)), ((), ())),
                              preferred_element_type=jnp.float32)
    contrib = contrib + rowsum * br_ref[0, r][None, :]

    @pl.when(r == 0)
    def _init():
        agg_ref[...] = contrib
        den_ref[...] = rowsum

    @pl.when(r > 0)
    def _acc():
        agg_ref[...] += contrib
        den_ref[...] += rowsum


def _body(adj_ref, x_ref, xown_ref, wr_ref, br_ref, w0_ref, b0_ref,
          out_ref, acache_ref, x1_ref, agg_ref, den_ref):
    l = pl.program_id(0)
    b = pl.program_id(1)
    n = pl.program_id(2)
    r = pl.program_id(3)
    idx = (b * NT + n) * R + r

    @pl.when(l == 0)
    def _layer0():
        adj_blk = adj_ref[0, 0]
        rowsum = jnp.sum(adj_blk, axis=1, keepdims=True)
        adj_bf = adj_blk.astype(jnp.bfloat16)
        acache_ref[idx] = adj_bf
        _compute(r, adj_bf, rowsum, x_ref[0], wr_ref, br_ref,
                 agg_ref, den_ref)

    @pl.when(l == 1)
    def _layer1():
        adj_bf = acache_ref[idx]
        rowsum = jnp.sum(adj_bf.astype(jnp.float32), axis=1, keepdims=True)
        _compute(r, adj_bf, rowsum, x1_ref[b], wr_ref, br_ref,
                 agg_ref, den_ref)

    @pl.when(r == R - 1)
    def _finish():
        x_own = lax.cond(l == 0,
                         lambda: xown_ref[0],
                         lambda: x1_ref[b, pl.ds(n * NTILE, NTILE)])
        h0 = lax.dot_general(x_own, w0_ref[0], (((1,), (1,)), ((), ())),
                             preferred_element_type=jnp.float32)
        h0 = h0 + b0_ref[0]
        denoms = den_ref[...] + 1.0
        out = jnp.maximum((agg_ref[...] + h0) / denoms, 0.0)
        out_ref[0, 0] = out

        @pl.when(l == 0)
        def _cache_x1():
            x1_ref[b, pl.ds(n * NTILE, NTILE)] = out.astype(jnp.bfloat16)


@jax.jit
def kernel(nodes, adj, W0, b0, Wr, br):
    bf = jnp.bfloat16
    L = W0.shape[0]
    xbf = nodes.astype(bf)

    outs = pl.pallas_call(
        _body,
        grid=(L, B, NT, R),
        in_specs=[
            pl.BlockSpec((1, 1, NTILE, N),
                         lambda l, b, n, r: (jnp.where(l == 0, b, 0),
                                             jnp.where(l == 0, r, 0),
                                             jnp.where(l == 0, n, 0), 0)),
            pl.BlockSpec((1, N, D),
                         lambda l, b, n, r: (jnp.where(l == 0, b, 0), 0, 0)),
            pl.BlockSpec((1, NTILE, D),
                         lambda l, b, n, r: (jnp.where(l == 0, b, 0),
                                             jnp.where(l == 0, n, 0), 0)),
            pl.BlockSpec((1, R, D, D), lambda l, b, n, r: (l, 0, 0, 0)),
            pl.BlockSpec((1, R, D), lambda l, b, n, r: (l, 0, 0)),
            pl.BlockSpec((1, D, D), lambda l, b, n, r: (l, 0, 0)),
            pl.BlockSpec((1, 1, D), lambda l, b, n, r: (l, 0, 0)),
        ],
        out_specs=pl.BlockSpec((1, 1, NTILE, D),
                               lambda l, b, n, r: (l, b, n, 0)),
        out_shape=jax.ShapeDtypeStruct((L, B, N, D), jnp.float32),
        scratch_shapes=[
            pltpu.VMEM((B * NT * R, NTILE, N), jnp.bfloat16),
            pltpu.VMEM((B, N, D), jnp.bfloat16),
            pltpu.VMEM((NTILE, D), jnp.float32),
            pltpu.VMEM((NTILE, 1), jnp.float32),
        ],
    )(adj, xbf, xbf, Wr.astype(bf), br, W0.astype(bf), b0[:, None, :])

    return tuple(outs[l] for l in range(L))


# speedup vs baseline: 1.1323x; 1.0222x over previous
import jax
import jax.numpy as jnp
from jax import lax
from jax.experimental import pallas as pl
from jax.experimental.pallas import tpu as pltpu

B, R, N, D = 4, 4, 1024, 256
NTILE = 256
NT = N // NTILE
L = 2


def _stage_s(r, adj_bf, x_full, scat_ref):
    s = jnp.dot(adj_bf, x_full, preferred_element_type=jnp.float32)
    sbf = s.astype(jnp.bfloat16)
    for k in range(R):
        @pl.when(r == k)
        def _():
            scat_ref[:, k * D:(k + 1) * D] = sbf


def _body(adj_ref, x_ref, xown_ref, wcat_ref, brm_ref, w0_ref, b0_ref,
          out0_ref, out1_ref,
          acache_ref, x1_ref, bias1_ref, den_ref,
          scat_ref, rsm_ref, dacc_ref):
    l = pl.program_id(0)
    b = pl.program_id(1)
    n = pl.program_id(2)
    r = pl.program_id(3)
    idx = (b * NT + n) * R + r
    bn = b * NT + n

    @pl.when(l == 0)
    def _layer0():
        adj_blk = adj_ref[0, 0]
        rowsum = jnp.sum(adj_blk, axis=1, keepdims=True)
        adj_bf = adj_blk.astype(jnp.bfloat16)
        acache_ref[idx] = adj_bf

        @pl.when(r == 0)
        def _():
            rsm_ref[...] = jnp.zeros((NTILE, 128), jnp.float32)
            dacc_ref[...] = rowsum

        for k in range(R):
            @pl.when(r == k)
            def _():
                rsm_ref[:, k:k + 1] = rowsum

        @pl.when(r > 0)
        def _():
            dacc_ref[...] += rowsum

        _stage_s(r, adj_bf, x_ref[0], scat_ref)

    @pl.when(l == 1)
    def _layer1():
        _stage_s(r, acache_ref[idx], x1_ref[b], scat_ref)

    @pl.when(r == R - 1)
    def _finish():
        agg = jnp.dot(scat_ref[...], wcat_ref[0],
                      preferred_element_type=jnp.float32)

        @pl.when(l == 0)
        def _():
            den_ref[bn] = dacc_ref[...] + 1.0
            rsm = rsm_ref[...]
            bias1_ref[bn] = jnp.dot(rsm, brm_ref[1],
                                    preferred_element_type=jnp.float32)

        bias = lax.cond(
            l == 0,
            lambda: jnp.dot(rsm_ref[...], brm_ref[0],
                            preferred_element_type=jnp.float32),
            lambda: bias1_ref[bn])

        x_own = lax.cond(l == 0,
                         lambda: xown_ref[0],
                         lambda: x1_ref[b, pl.ds(n * NTILE, NTILE)])
        h0 = lax.dot_general(x_own, w0_ref[0], (((1,), (1,)), ((), ())),
                             preferred_element_type=jnp.float32)
        out = jnp.maximum((agg + bias + h0 + b0_ref[0]) / den_ref[bn], 0.0)

        @pl.when(l == 0)
        def _():
            out0_ref[0] = out
            x1_ref[b, pl.ds(n * NTILE, NTILE)] = out.astype(jnp.bfloat16)

        @pl.when(l == 1)
        def _():
            out1_ref[0] = out


@jax.jit
def kernel(nodes, adj, W0, b0, Wr, br):
    bf = jnp.bfloat16
    xbf = nodes.astype(bf)
    wcat = Wr.transpose(0, 1, 3, 2).reshape(L, R * D, D).astype(bf)
    brm = jnp.zeros((L, 128, D), jnp.float32).at[:, :R, :].set(br)

    out0, out1 = pl.pallas_call(
        _body,
        grid=(L, B, NT, R),
        in_specs=[
            pl.BlockSpec((1, 1, NTILE, N),
                         lambda l, b, n, r: (jnp.where(l == 0, b, 0),
                                             jnp.where(l == 0, r, 0),
                                             jnp.where(l == 0, n, 0), 0)),
            pl.BlockSpec((1, N, D),
                         lambda l, b, n, r: (jnp.where(l == 0, b, 0), 0, 0)),
            pl.BlockSpec((1, NTILE, D),
                         lambda l, b, n, r: (jnp.where(l == 0, b, 0),
                                             jnp.where(l == 0, n, 0), 0)),
            pl.BlockSpec((1, R * D, D), lambda l, b, n, r: (l, 0, 0)),
            pl.BlockSpec((L, 128, D), lambda l, b, n, r: (0, 0, 0)),
            pl.BlockSpec((1, D, D), lambda l, b, n, r: (l, 0, 0)),
            pl.BlockSpec((1, 1, D), lambda l, b, n, r: (l, 0, 0)),
        ],
        out_specs=[
            pl.BlockSpec((1, NTILE, D),
                         lambda l, b, n, r: (jnp.where(l == 0, b, B - 1),
                                             jnp.where(l == 0, n, NT - 1), 0)),
            pl.BlockSpec((1, NTILE, D),
                         lambda l, b, n, r: (jnp.where(l == 0, 0, b),
                                             jnp.where(l == 0, 0, n), 0)),
        ],
        out_shape=[
            jax.ShapeDtypeStruct((B, N, D), jnp.float32),
            jax.ShapeDtypeStruct((B, N, D), jnp.float32),
        ],
        scratch_shapes=[
            pltpu.VMEM((B * NT * R, NTILE, N), jnp.bfloat16),
            pltpu.VMEM((B, N, D), jnp.bfloat16),
            pltpu.VMEM((B * NT, NTILE, D), jnp.float32),
            pltpu.VMEM((B * NT, NTILE, 1), jnp.float32),
            pltpu.VMEM((NTILE, R * D), jnp.bfloat16),
            pltpu.VMEM((NTILE, 128), jnp.float32),
            pltpu.VMEM((NTILE, 1), jnp.float32),
        ],
    )(adj, xbf, xbf, wcat, brm, W0.astype(bf), b0[:, None, :])
    return (out0, out1)


# speedup vs baseline: 1.5627x; 1.3802x over previous
import jax
import jax.numpy as jnp
from jax import lax
from jax.experimental import pallas as pl
from jax.experimental.pallas import tpu as pltpu

B, R, N, D = 4, 4, 1024, 256
NTILE = 512
NT = N // NTILE
L = 2


def _stage_s(r, adj_bf, x_full, scat_ref):
    s = jnp.dot(adj_bf, x_full, preferred_element_type=jnp.float32)
    sbf = s.astype(jnp.bfloat16)
    for k in range(R):
        @pl.when(r == k)
        def _():
            scat_ref[:, k * D:(k + 1) * D] = sbf


def _body(adj_ref, x_ref, xown_ref, wcat_ref, brm_ref, w0_ref, b0_ref,
          out0_ref, out1_ref,
          acache_ref, x1_ref, bias1_ref, den_ref,
          scat_ref, rsm_ref, dacc_ref):
    l = pl.program_id(0)
    b = pl.program_id(1)
    n = pl.program_id(2)
    r = pl.program_id(3)
    idx = (b * NT + n) * R + r
    bn = b * NT + n

    @pl.when(l == 0)
    def _layer0():
        adj_blk = adj_ref[0, 0]
        rowsum = jnp.sum(adj_blk, axis=1, keepdims=True)
        adj_bf = adj_blk.astype(jnp.bfloat16)
        acache_ref[idx] = adj_bf

        @pl.when(r == 0)
        def _():
            rsm_ref[...] = jnp.zeros((NTILE, 128), jnp.float32)
            dacc_ref[...] = rowsum

        for k in range(R):
            @pl.when(r == k)
            def _():
                rsm_ref[:, k:k + 1] = rowsum

        @pl.when(r > 0)
        def _():
            dacc_ref[...] += rowsum

        _stage_s(r, adj_bf, x_ref[0], scat_ref)

    @pl.when(l == 1)
    def _layer1():
        _stage_s(r, acache_ref[idx], x1_ref[b], scat_ref)

    @pl.when(r == R - 1)
    def _finish():
        agg = jnp.dot(scat_ref[...], wcat_ref[0],
                      preferred_element_type=jnp.float32)

        @pl.when(l == 0)
        def _():
            den_ref[bn] = dacc_ref[...] + 1.0
            rsm = rsm_ref[...]
            bias1_ref[bn] = jnp.dot(rsm, brm_ref[1],
                                    preferred_element_type=jnp.float32)

        bias = lax.cond(
            l == 0,
            lambda: jnp.dot(rsm_ref[...], brm_ref[0],
                            preferred_element_type=jnp.float32),
            lambda: bias1_ref[bn])

        x_own = lax.cond(l == 0,
                         lambda: xown_ref[0],
                         lambda: x1_ref[b, pl.ds(n * NTILE, NTILE)])
        h0 = lax.dot_general(x_own, w0_ref[0], (((1,), (1,)), ((), ())),
                             preferred_element_type=jnp.float32)
        out = jnp.maximum((agg + bias + h0 + b0_ref[0]) / den_ref[bn], 0.0)

        @pl.when(l == 0)
        def _():
            out0_ref[0] = out
            x1_ref[b, pl.ds(n * NTILE, NTILE)] = out.astype(jnp.bfloat16)

        @pl.when(l == 1)
        def _():
            out1_ref[0] = out


@jax.jit
def kernel(nodes, adj, W0, b0, Wr, br):
    bf = jnp.bfloat16
    xbf = nodes.astype(bf)
    wcat = Wr.transpose(0, 1, 3, 2).reshape(L, R * D, D).astype(bf)
    brm = jnp.zeros((L, 128, D), jnp.float32).at[:, :R, :].set(br)

    out0, out1 = pl.pallas_call(
        _body,
        grid=(L, B, NT, R),
        in_specs=[
            pl.BlockSpec((1, 1, NTILE, N),
                         lambda l, b, n, r: (jnp.where(l == 0, b, 0),
                                             jnp.where(l == 0, r, 0),
                                             jnp.where(l == 0, n, 0), 0)),
            pl.BlockSpec((1, N, D),
                         lambda l, b, n, r: (jnp.where(l == 0, b, 0), 0, 0)),
            pl.BlockSpec((1, NTILE, D),
                         lambda l, b, n, r: (jnp.where(l == 0, b, 0),
                                             jnp.where(l == 0, n, 0), 0)),
            pl.BlockSpec((1, R * D, D), lambda l, b, n, r: (l, 0, 0)),
            pl.BlockSpec((L, 128, D), lambda l, b, n, r: (0, 0, 0)),
            pl.BlockSpec((1, D, D), lambda l, b, n, r: (l, 0, 0)),
            pl.BlockSpec((1, 1, D), lambda l, b, n, r: (l, 0, 0)),
        ],
        out_specs=[
            pl.BlockSpec((1, NTILE, D),
                         lambda l, b, n, r: (jnp.where(l == 0, b, B - 1),
                                             jnp.where(l == 0, n, NT - 1), 0)),
            pl.BlockSpec((1, NTILE, D),
                         lambda l, b, n, r: (jnp.where(l == 0, 0, b),
                                             jnp.where(l == 0, 0, n), 0)),
        ],
        out_shape=[
            jax.ShapeDtypeStruct((B, N, D), jnp.float32),
            jax.ShapeDtypeStruct((B, N, D), jnp.float32),
        ],
        scratch_shapes=[
            pltpu.VMEM((B * NT * R, NTILE, N), jnp.bfloat16),
            pltpu.VMEM((B, N, D), jnp.bfloat16),
            pltpu.VMEM((B * NT, NTILE, D), jnp.float32),
            pltpu.VMEM((B * NT, NTILE, 1), jnp.float32),
            pltpu.VMEM((NTILE, R * D), jnp.bfloat16),
            pltpu.VMEM((NTILE, 128), jnp.float32),
            pltpu.VMEM((NTILE, 1), jnp.float32),
        ],
    )(adj, xbf, xbf, wcat, brm, W0.astype(bf), b0[:, None, :])
    return (out0, out1)
